# baseline (speedup 1.0000x reference)
import jax
import jax.numpy as jnp
from jax import lax
from jax.experimental import pallas as pl
from jax.experimental.pallas import tpu as pltpu
from jax.experimental.pallas import tpu_sc as plsc

N = 10000
E = 160000
D = 256
D_EDGE = 16

NC = 2
NS = 16
NW = NC * NS
E_PAD = 163840
N_PAD = 10240
ROWS = N_PAD // NW
CH = 2048
NCHUNK = E_PAD // CH

_SC_MESH = dict(core_axis_name="c", subcore_axis_name="s")
_SC_PARAMS = pltpu.CompilerParams(needs_layout_passes=False)


_BN = 1000


def _xlin_body(x_ref, w_ref, b_ref, out_ref):
    out_ref[...] = (
        jnp.dot(x_ref[...], w_ref[...], preferred_element_type=jnp.float32)
        + b_ref[...]
    )


def _xlin(x, W_conv, b_conv):
    return pl.pallas_call(
        _xlin_body,
        grid=(N // _BN,),
        in_specs=[
            pl.BlockSpec((_BN, D), lambda i: (i, 0)),
            pl.BlockSpec((D, D), lambda i: (0, 0)),
            pl.BlockSpec((1, D), lambda i: (0, 0)),
        ],
        out_specs=pl.BlockSpec((_BN, D), lambda i: (i, 0)),
        out_shape=jax.ShapeDtypeStruct((N, D), jnp.float32),
    )(x, W_conv, b_conv.reshape(1, D))


_BE = 2048


def _ef_body(ea_ref, w_ref, b_ref, ef_ref, ew_ref):
    i = pl.program_id(0)
    ea = ea_ref[...]
    rows = i * _BE + lax.broadcasted_iota(jnp.int32, (_BE, 1), 0)
    live = rows < E
    ef_ref[...] = jnp.where(
        live,
        jnp.maximum(
            jnp.dot(ea, w_ref[...], preferred_element_type=jnp.float32)
            + b_ref[...],
            0.0,
        ),
        0.0,
    )
    ew_ref[...] = jnp.sqrt(jnp.sum(ea * ea, axis=1, keepdims=True))


def _ef_and_weight(edge_attr_pad, W_ep, b_ep):
    return pl.pallas_call(
        _ef_body,
        grid=(E_PAD // _BE,),
        in_specs=[
            pl.BlockSpec((_BE, D_EDGE), lambda i: (i, 0)),
            pl.BlockSpec((D_EDGE, D), lambda i: (0, 0)),
            pl.BlockSpec((1, D), lambda i: (0, 0)),
        ],
        out_specs=[
            pl.BlockSpec((_BE, D), lambda i: (i, 0)),
            pl.BlockSpec((_BE, 1), lambda i: (i, 0)),
        ],
        out_shape=[
            jax.ShapeDtypeStruct((E_PAD, D), jnp.float32),
            jax.ShapeDtypeStruct((E_PAD, 1), jnp.float32),
        ],
    )(edge_attr_pad, W_ep, b_ep.reshape(1, D))


def _deg_body(dst_hbm, w_hbm, out_hbm, dstc, wc, pdst, pw, acc):
    c = lax.axis_index("c")
    s = lax.axis_index("s")
    wid = c * NS + s
    lo = wid * ROWS

    for r in range(ROWS):
        acc[r, pl.ds(0, 16)] = jnp.zeros((16,), jnp.float32)

    iota = lax.iota(jnp.int32, 16)
    lane0 = iota == 0
    allm = iota >= 0
    zf = jnp.zeros((16,), jnp.float32)
    zi = jnp.zeros((16,), jnp.int32)

    def chunk(j, _):
        base = j * CH
        pltpu.sync_copy(dst_hbm.at[pl.ds(base, CH)], dstc)
        pltpu.sync_copy(w_hbm.at[pl.ds(base, CH)], wc)
        m = jnp.int32(0)
        for g in range(CH // 16):
            d16 = dstc[pl.ds(g * 16, 16)]
            w16 = wc[pl.ds(g * 16, 16)]
            loc = d16 - lo
            mask = (loc >= 0) & (loc < ROWS)
            plsc.store_compressed(pdst.at[pl.ds(m, 16)], loc, mask=mask)
            plsc.store_compressed(pw.at[pl.ds(m, 16)], w16, mask=mask)
            m = m + plsc.all_reduce_population_count(mask)[0]
        plsc.store_compressed(pdst.at[pl.ds(m, 16)], zi, mask=allm)
        plsc.store_compressed(pw.at[pl.ds(m, 16)], zf, mask=allm)

        def batch(b, _):
            dvec = pdst[pl.ds(b * 16, 16)]
            wvec = pw[pl.ds(b * 16, 16)]
            for l in range(16):
                plsc.addupdate(acc.at[dvec[l], pl.ds(0, 16)],
                               jnp.where(lane0, wvec[l], 0.0))
            return 0

        lax.fori_loop(0, (m + 15) // 16, batch, 0)
        return 0

    lax.fori_loop(0, NCHUNK, chunk, 0)
    pltpu.sync_copy(acc, out_hbm.at[pl.ds(lo, ROWS)])


def _sc_degree(dst_pad, w_pad):
    kfn = pl.kernel(
        _deg_body,
        out_type=jax.ShapeDtypeStruct((N_PAD, 16), jnp.float32),
        mesh=plsc.VectorSubcoreMesh(**_SC_MESH),
        compiler_params=_SC_PARAMS,
        scratch_types=[
            pltpu.VMEM((CH,), jnp.int32),
            pltpu.VMEM((CH,), jnp.float32),
            pltpu.VMEM((CH + 16,), jnp.int32),
            pltpu.VMEM((CH + 16,), jnp.float32),
            pltpu.VMEM((ROWS, 16), jnp.float32),
        ],
    )
    return kfn(dst_pad, w_pad)


def _dinv_body(d_ref, out_ref):
    out_ref[...] = lax.rsqrt(1.0 + jnp.sum(d_ref[...], axis=1, keepdims=True))


def _dinv(deg16):
    return pl.pallas_call(
        _dinv_body,
        grid=(1,),
        in_specs=[pl.BlockSpec((N_PAD, 16), lambda i: (0, 0))],
        out_specs=pl.BlockSpec((N_PAD, 1), lambda i: (0, 0)),
        out_shape=jax.ShapeDtypeStruct((N_PAD, 1), jnp.float32),
    )(deg16).reshape(N_PAD)


def _xconv_body(src_hbm, dst_hbm, w_hbm, dinv_hbm, xlin_hbm, out_hbm,
                srcc, dstc, wc, dinvv, psrc, pdst, pnorm, rowb, acc, gsem,
                lsem):
    c = lax.axis_index("c")
    s = lax.axis_index("s")
    wid = c * NS + s
    lo = wid * ROWS

    def zrow(r, _):
        for v in range(16):
            acc[r, pl.ds(v * 16, 16)] = jnp.zeros((16,), jnp.float32)
        return 0

    lax.fori_loop(0, ROWS, zrow, 0)

    pltpu.sync_copy(dinv_hbm, dinvv)

    iota = lax.iota(jnp.int32, 16)
    allm = iota >= 0
    zf = jnp.zeros((16,), jnp.float32)
    zi = jnp.zeros((16,), jnp.int32)

    def chunk(j, _):
        base = j * CH
        d1 = pltpu.async_copy(src_hbm.at[pl.ds(base, CH)], srcc, lsem)
        d2 = pltpu.async_copy(dst_hbm.at[pl.ds(base, CH)], dstc, lsem)
        d3 = pltpu.async_copy(w_hbm.at[pl.ds(base, CH)], wc, lsem)
        d1.wait()
        d2.wait()
        d3.wait()
        m = jnp.int32(0)
        for g in range(CH // 16):
            s16 = srcc[pl.ds(g * 16, 16)]
            d16 = dstc[pl.ds(g * 16, 16)]
            w16 = wc[pl.ds(g * 16, 16)]
            loc = d16 - lo
            mask = (loc >= 0) & (loc < ROWS)
            dvs = plsc.load_gather(dinvv, [s16])
            dvd = plsc.load_gather(dinvv, [d16])
            norm16 = dvs * w16 * dvd
            plsc.store_compressed(psrc.at[pl.ds(m, 16)], s16, mask=mask)
            plsc.store_compressed(pdst.at[pl.ds(m, 16)], loc, mask=mask)
            plsc.store_compressed(pnorm.at[pl.ds(m, 16)], norm16, mask=mask)
            m = m + plsc.all_reduce_population_count(mask)[0]
        for t in range(3):
            plsc.store_compressed(psrc.at[pl.ds(m + t * 16, 16)], zi,
                                  mask=allm)
            plsc.store_compressed(pdst.at[pl.ds(m + t * 16, 16)], zi,
                                  mask=allm)
            plsc.store_compressed(pnorm.at[pl.ds(m + t * 16, 16)], zf,
                                  mask=allm)

        def wave(wv, _):
            o48 = wv * 48
            pltpu.async_copy(xlin_hbm.at[psrc.at[pl.ds(o48, 48)]], rowb,
                             gsem).wait()
            for bb in range(3):
                dvec = pdst[pl.ds(o48 + bb * 16, 16)]
                nvec = pnorm[pl.ds(o48 + bb * 16, 16)]
                for l in range(16):
                    dloc = dvec[l]
                    ns = nvec[l]
                    for v in range(16):
                        plsc.addupdate(acc.at[dloc, pl.ds(v * 16, 16)],
                                       rowb[bb * 16 + l, pl.ds(v * 16, 16)]
                                       * ns)
            return 0

        lax.fori_loop(0, (m + 47) // 48, wave, 0)
        return 0

    lax.fori_loop(0, NCHUNK, chunk, 0)
    pltpu.sync_copy(acc, out_hbm.at[pl.ds(lo, ROWS)])


def _sc_xconv(src_pad, dst_pad, w_pad, dinv, x_lin):
    kfn = pl.kernel(
        _xconv_body,
        out_type=jax.ShapeDtypeStruct((N_PAD, D), jnp.float32),
        mesh=plsc.VectorSubcoreMesh(**_SC_MESH),
        compiler_params=_SC_PARAMS,
        scratch_types=[
            pltpu.VMEM((CH,), jnp.int32),
            pltpu.VMEM((CH,), jnp.int32),
            pltpu.VMEM((CH,), jnp.float32),
            pltpu.VMEM((N_PAD,), jnp.float32),
            pltpu.VMEM((CH + 48,), jnp.int32),
            pltpu.VMEM((CH + 48,), jnp.int32),
            pltpu.VMEM((CH + 48,), jnp.float32),
            pltpu.VMEM((48, D), jnp.float32),
            pltpu.VMEM((ROWS, D), jnp.float32),
            pltpu.SemaphoreType.DMA,
            pltpu.SemaphoreType.DMA,
        ],
    )
    return kfn(src_pad, dst_pad, w_pad, dinv, x_lin)


def _efagg_body(src_hbm, ef_hbm, out_hbm, srcc, psrc, peid, rowb, acc, gsem):
    c = lax.axis_index("c")
    s = lax.axis_index("s")
    wid = c * NS + s
    lo = wid * ROWS

    def zrow(r, _):
        for v in range(16):
            acc[r, pl.ds(v * 16, 16)] = jnp.zeros((16,), jnp.float32)
        return 0

    lax.fori_loop(0, ROWS + 1, zrow, 0)

    iota = lax.iota(jnp.int32, 16)
    allm = iota >= 0
    zi = jnp.zeros((16,), jnp.int32)

    def chunk(j, _):
        base = j * CH
        pltpu.sync_copy(src_hbm.at[pl.ds(base, CH)], srcc)
        m = jnp.int32(0)
        for g in range(CH // 16):
            s16 = srcc[pl.ds(g * 16, 16)]
            loc = s16 - lo
            mask = (loc >= 0) & (loc < ROWS)
            eid16 = base + g * 16 + iota
            plsc.store_compressed(psrc.at[pl.ds(m, 16)], loc, mask=mask)
            plsc.store_compressed(peid.at[pl.ds(m, 16)], eid16, mask=mask)
            m = m + plsc.all_reduce_population_count(mask)[0]
        for t in range(3):
            plsc.store_compressed(psrc.at[pl.ds(m + t * 16, 16)], zi,
                                  mask=allm)
            plsc.store_compressed(peid.at[pl.ds(m + t * 16, 16)], zi,
                                  mask=allm)

        def wave(wv, _):
            o48 = wv * 48
            pltpu.async_copy(ef_hbm.at[peid.at[pl.ds(o48, 48)]], rowb,
                             gsem).wait()
            for bb in range(3):
                svec = psrc[pl.ds(o48 + bb * 16, 16)]
                live = (o48 + bb * 16 + iota) < m
                svec = jnp.where(live, svec, ROWS)
                for l in range(16):
                    sloc = svec[l]
                    for v in range(16):
                        plsc.addupdate(acc.at[sloc, pl.ds(v * 16, 16)],
                                       rowb[bb * 16 + l, pl.ds(v * 16, 16)])
            return 0

        lax.fori_loop(0, (m + 47) // 48, wave, 0)
        return 0

    lax.fori_loop(0, NCHUNK, chunk, 0)
    pltpu.sync_copy(acc.at[pl.ds(0, ROWS)], out_hbm.at[pl.ds(lo, ROWS)])


def _sc_efagg(src_pad, ef):
    kfn = pl.kernel(
        _efagg_body,
        out_type=jax.ShapeDtypeStruct((N_PAD, D), jnp.float32),
        mesh=plsc.VectorSubcoreMesh(**_SC_MESH),
        compiler_params=_SC_PARAMS,
        scratch_types=[
            pltpu.VMEM((CH,), jnp.int32),
            pltpu.VMEM((CH + 48,), jnp.int32),
            pltpu.VMEM((CH + 48,), jnp.int32),
            pltpu.VMEM((48, D), jnp.float32),
            pltpu.VMEM((ROWS + 1, D), jnp.float32),
            pltpu.SemaphoreType.DMA,
        ],
    )
    return kfn(src_pad, ef)


def _epilogue_body(xc_ref, ea_ref, xlin_ref, dinv_ref, wg1_ref, wg2_ref,
                   bg_ref, gamma_ref, beta_ref, out_ref):
    dinv = dinv_ref[...]
    x_conv = xc_ref[...] + (dinv * dinv) * xlin_ref[...]
    ef_agg = ea_ref[...]
    gate = jax.nn.sigmoid(
        jnp.dot(x_conv, wg1_ref[...], preferred_element_type=jnp.float32)
        + jnp.dot(ef_agg, wg2_ref[...], preferred_element_type=jnp.float32)
        + bg_ref[...]
    )
    out = gate * x_conv + (1.0 - gate) * ef_agg
    mu = jnp.mean(out, axis=-1, keepdims=True)
    var = jnp.mean((out - mu) ** 2, axis=-1, keepdims=True)
    out = (out - mu) * lax.rsqrt(var + 1e-5) * gamma_ref[...] + beta_ref[...]
    out_ref[...] = jnp.maximum(out, 0.0)


def _epilogue(xc, ea, x_lin, dinv, W_gate, b_gate, ln_gamma, ln_beta):
    row_spec = pl.BlockSpec((_BN, D), lambda i: (i, 0))
    return pl.pallas_call(
        _epilogue_body,
        grid=(N // _BN,),
        in_specs=[
            row_spec, row_spec, row_spec,
            pl.BlockSpec((_BN, 1), lambda i: (i, 0)),
            pl.BlockSpec((D, D), lambda i: (0, 0)),
            pl.BlockSpec((D, D), lambda i: (0, 0)),
            pl.BlockSpec((1, D), lambda i: (0, 0)),
            pl.BlockSpec((1, D), lambda i: (0, 0)),
            pl.BlockSpec((1, D), lambda i: (0, 0)),
        ],
        out_specs=pl.BlockSpec((_BN, D), lambda i: (i, 0)),
        out_shape=jax.ShapeDtypeStruct((N, D), jnp.float32),
    )(xc, ea, x_lin, dinv.reshape(N_PAD, 1)[:N], W_gate[:D], W_gate[D:],
      b_gate.reshape(1, D), ln_gamma.reshape(1, D), ln_beta.reshape(1, D))


def kernel(x, edge_index, edge_attr, W_conv, b_conv, W_ep, b_ep, W_gate,
           b_gate, ln_gamma, ln_beta):
    pad = E_PAD - E
    src_pad = jnp.concatenate([edge_index[0], jnp.zeros((pad,), jnp.int32)])
    dst_pad = jnp.concatenate([edge_index[1], jnp.zeros((pad,), jnp.int32)])
    ea_pad = jnp.concatenate(
        [edge_attr, jnp.zeros((pad, D_EDGE), jnp.float32)])

    x_lin = _xlin(x, W_conv, b_conv)
    ef, edge_weight = _ef_and_weight(ea_pad, W_ep, b_ep)
    w_pad = edge_weight.reshape(E_PAD)

    deg16 = _sc_degree(dst_pad, w_pad)
    dinv = _dinv(deg16)

    xc = _sc_xconv(src_pad, dst_pad, w_pad, dinv, x_lin)
    ea = _sc_efagg(src_pad, ef)

    return _epilogue(xc[:N], ea[:N], x_lin, dinv, W_gate, b_gate,
                     ln_gamma, ln_beta)


# speedup vs baseline: 1.5703x; 1.5703x over previous
import jax
import jax.numpy as jnp
from jax import lax
from jax.experimental import pallas as pl
from jax.experimental.pallas import tpu as pltpu
from jax.experimental.pallas import tpu_sc as plsc

N = 10000
E = 160000
D = 256
D_EDGE = 16

NC = 2
NS = 16
NW = NC * NS
E_PAD = 163840
N_PAD = 10240
ROWS = N_PAD // NW
CH = 1024
NCHUNK = E_PAD // CH

_SC_MESH = dict(core_axis_name="c", subcore_axis_name="s")
_SC_PARAMS = pltpu.CompilerParams(needs_layout_passes=False)


_BN = 1000


def _xlin_body(x_ref, w_ref, b_ref, out_ref):
    out_ref[...] = (
        jnp.dot(x_ref[...], w_ref[...], preferred_element_type=jnp.float32)
        + b_ref[...]
    )


def _xlin(x, W_conv, b_conv):
    return pl.pallas_call(
        _xlin_body,
        grid=(N // _BN,),
        in_specs=[
            pl.BlockSpec((_BN, D), lambda i: (i, 0)),
            pl.BlockSpec((D, D), lambda i: (0, 0)),
            pl.BlockSpec((1, D), lambda i: (0, 0)),
        ],
        out_specs=pl.BlockSpec((_BN, D), lambda i: (i, 0)),
        out_shape=jax.ShapeDtypeStruct((N, D), jnp.float32),
    )(x, W_conv, b_conv.reshape(1, D))


_BE = 2048


def _ef_body(ea_ref, w_ref, b_ref, ef_ref, ew_ref):
    i = pl.program_id(0)
    ea = ea_ref[...]
    rows = i * _BE + lax.broadcasted_iota(jnp.int32, (_BE, 1), 0)
    live = rows < E
    ef_ref[...] = jnp.where(
        live,
        jnp.maximum(
            jnp.dot(ea, w_ref[...], preferred_element_type=jnp.float32)
            + b_ref[...],
            0.0,
        ),
        0.0,
    )
    ew_ref[...] = jnp.sqrt(jnp.sum(ea * ea, axis=1, keepdims=True))


def _ef_and_weight(edge_attr_pad, W_ep, b_ep):
    return pl.pallas_call(
        _ef_body,
        grid=(E_PAD // _BE,),
        in_specs=[
            pl.BlockSpec((_BE, D_EDGE), lambda i: (i, 0)),
            pl.BlockSpec((D_EDGE, D), lambda i: (0, 0)),
            pl.BlockSpec((1, D), lambda i: (0, 0)),
        ],
        out_specs=[
            pl.BlockSpec((_BE, D), lambda i: (i, 0)),
            pl.BlockSpec((_BE, 1), lambda i: (i, 0)),
        ],
        out_shape=[
            jax.ShapeDtypeStruct((E_PAD, D), jnp.float32),
            jax.ShapeDtypeStruct((E_PAD, 1), jnp.float32),
        ],
    )(edge_attr_pad, W_ep, b_ep.reshape(1, D))


def _deg_body(dst_hbm, w_hbm, out_hbm, dstc, wc, pdst, pw, acc):
    c = lax.axis_index("c")
    s = lax.axis_index("s")
    wid = c * NS + s
    lo = wid * ROWS

    for r in range(ROWS):
        acc[r, pl.ds(0, 16)] = jnp.zeros((16,), jnp.float32)

    iota = lax.iota(jnp.int32, 16)
    lane0 = iota == 0
    allm = iota >= 0
    zf = jnp.zeros((16,), jnp.float32)
    zi = jnp.zeros((16,), jnp.int32)

    def chunk(j, _):
        base = j * CH
        pltpu.sync_copy(dst_hbm.at[pl.ds(base, CH)], dstc)
        pltpu.sync_copy(w_hbm.at[pl.ds(base, CH)], wc)
        m = jnp.int32(0)
        for g in range(CH // 16):
            d16 = dstc[pl.ds(g * 16, 16)]
            w16 = wc[pl.ds(g * 16, 16)]
            loc = d16 - lo
            mask = (loc >= 0) & (loc < ROWS)
            plsc.store_compressed(pdst.at[pl.ds(m, 16)], loc, mask=mask)
            plsc.store_compressed(pw.at[pl.ds(m, 16)], w16, mask=mask)
            m = m + plsc.all_reduce_population_count(mask)[0]
        plsc.store_compressed(pdst.at[pl.ds(m, 16)], zi, mask=allm)
        plsc.store_compressed(pw.at[pl.ds(m, 16)], zf, mask=allm)

        def batch(b, _):
            dvec = pdst[pl.ds(b * 16, 16)]
            wvec = pw[pl.ds(b * 16, 16)]
            for l in range(16):
                plsc.addupdate(acc.at[dvec[l], pl.ds(0, 16)],
                               jnp.where(lane0, wvec[l], 0.0))
            return 0

        lax.fori_loop(0, (m + 15) // 16, batch, 0)
        return 0

    lax.fori_loop(0, NCHUNK, chunk, 0)
    pltpu.sync_copy(acc, out_hbm.at[pl.ds(lo, ROWS)])


def _sc_degree(dst_pad, w_pad):
    kfn = pl.kernel(
        _deg_body,
        out_type=jax.ShapeDtypeStruct((N_PAD, 16), jnp.float32),
        mesh=plsc.VectorSubcoreMesh(**_SC_MESH),
        compiler_params=_SC_PARAMS,
        scratch_types=[
            pltpu.VMEM((CH,), jnp.int32),
            pltpu.VMEM((CH,), jnp.float32),
            pltpu.VMEM((CH + 16,), jnp.int32),
            pltpu.VMEM((CH + 16,), jnp.float32),
            pltpu.VMEM((ROWS, 16), jnp.float32),
        ],
    )
    return kfn(dst_pad, w_pad)


def _dinv_body(d_ref, out_ref):
    out_ref[...] = lax.rsqrt(1.0 + jnp.sum(d_ref[...], axis=1, keepdims=True))


def _dinv(deg16):
    return pl.pallas_call(
        _dinv_body,
        grid=(1,),
        in_specs=[pl.BlockSpec((N_PAD, 16), lambda i: (0, 0))],
        out_specs=pl.BlockSpec((N_PAD, 1), lambda i: (0, 0)),
        out_shape=jax.ShapeDtypeStruct((N_PAD, 1), jnp.float32),
    )(deg16).reshape(N_PAD)


def _xconv_body(src_hbm, dst_hbm, w_hbm, dinv_hbm, xlin_hbm, out_hbm,
                srcc, dstc, wc, dinvv, psrc, pdst, pnorm, rowb, acc, gsem,
                lsem):
    c = lax.axis_index("c")
    s = lax.axis_index("s")
    wid = c * NS + s
    lo = wid * ROWS

    def zrow(r, _):
        for v in range(16):
            acc[r, pl.ds(v * 16, 16)] = jnp.zeros((16,), jnp.float32)
        return 0

    lax.fori_loop(0, ROWS, zrow, 0)

    pltpu.sync_copy(dinv_hbm, dinvv)

    iota = lax.iota(jnp.int32, 16)
    allm = iota >= 0
    zf = jnp.zeros((16,), jnp.float32)
    zi = jnp.zeros((16,), jnp.int32)

    def chunk(j, _):
        base = j * CH
        d1 = pltpu.async_copy(src_hbm.at[pl.ds(base, CH)], srcc, lsem)
        d2 = pltpu.async_copy(dst_hbm.at[pl.ds(base, CH)], dstc, lsem)
        d3 = pltpu.async_copy(w_hbm.at[pl.ds(base, CH)], wc, lsem)
        d1.wait()
        d2.wait()
        d3.wait()
        m = jnp.int32(0)
        for g in range(CH // 16):
            s16 = srcc[pl.ds(g * 16, 16)]
            d16 = dstc[pl.ds(g * 16, 16)]
            w16 = wc[pl.ds(g * 16, 16)]
            loc = d16 - lo
            mask = (loc >= 0) & (loc < ROWS)
            dvs = plsc.load_gather(dinvv, [s16])
            dvd = plsc.load_gather(dinvv, [d16])
            norm16 = dvs * w16 * dvd
            plsc.store_compressed(psrc.at[pl.ds(m, 16)], s16, mask=mask)
            plsc.store_compressed(pdst.at[pl.ds(m, 16)], loc, mask=mask)
            plsc.store_compressed(pnorm.at[pl.ds(m, 16)], norm16, mask=mask)
            m = m + plsc.all_reduce_population_count(mask)[0]
        plsc.store_compressed(psrc.at[pl.ds(m, 16)], zi, mask=allm)
        plsc.store_compressed(pdst.at[pl.ds(m, 16)], zi, mask=allm)
        plsc.store_compressed(pnorm.at[pl.ds(m, 16)], zf, mask=allm)

        def batch(b, _):
            pltpu.async_copy(xlin_hbm.at[psrc.at[pl.ds(b * 16, 16)]], rowb,
                             gsem).wait()
            dvec = pdst[pl.ds(b * 16, 16)]
            nvec = pnorm[pl.ds(b * 16, 16)]
            for l in range(16):
                dloc = dvec[l]
                ns = nvec[l]
                for v in range(16):
                    plsc.addupdate(acc.at[dloc, pl.ds(v * 16, 16)],
                                   rowb[l, pl.ds(v * 16, 16)] * ns)
            return 0

        lax.fori_loop(0, (m + 15) // 16, batch, 0)
        return 0

    lax.fori_loop(0, NCHUNK, chunk, 0)
    pltpu.sync_copy(acc, out_hbm.at[pl.ds(lo, ROWS)])


def _sc_xconv(src_pad, dst_pad, w_pad, dinv, x_lin):
    kfn = pl.kernel(
        _xconv_body,
        out_type=jax.ShapeDtypeStruct((N_PAD, D), jnp.float32),
        mesh=plsc.VectorSubcoreMesh(**_SC_MESH),
        compiler_params=_SC_PARAMS,
        scratch_types=[
            pltpu.VMEM((CH,), jnp.int32),
            pltpu.VMEM((CH,), jnp.int32),
            pltpu.VMEM((CH,), jnp.float32),
            pltpu.VMEM((N_PAD,), jnp.float32),
            pltpu.VMEM((CH + 48,), jnp.int32),
            pltpu.VMEM((CH + 48,), jnp.int32),
            pltpu.VMEM((CH + 48,), jnp.float32),
            pltpu.VMEM((16, D), jnp.float32),
            pltpu.VMEM((ROWS, D), jnp.float32),
            pltpu.SemaphoreType.DMA,
            pltpu.SemaphoreType.DMA,
        ],
    )
    return kfn(src_pad, dst_pad, w_pad, dinv, x_lin)


def _efagg_body(src_hbm, ef_hbm, out_hbm, srcc, psrc, peid, rowb, acc, gsem):
    c = lax.axis_index("c")
    s = lax.axis_index("s")
    wid = c * NS + s
    lo = wid * ROWS

    def zrow(r, _):
        for v in range(16):
            acc[r, pl.ds(v * 16, 16)] = jnp.zeros((16,), jnp.float32)
        return 0

    lax.fori_loop(0, ROWS + 1, zrow, 0)

    iota = lax.iota(jnp.int32, 16)
    allm = iota >= 0
    zi = jnp.zeros((16,), jnp.int32)

    def chunk(j, _):
        base = j * CH
        pltpu.sync_copy(src_hbm.at[pl.ds(base, CH)], srcc)
        m = jnp.int32(0)
        for g in range(CH // 16):
            s16 = srcc[pl.ds(g * 16, 16)]
            loc = s16 - lo
            mask = (loc >= 0) & (loc < ROWS)
            eid16 = base + g * 16 + iota
            plsc.store_compressed(psrc.at[pl.ds(m, 16)], loc, mask=mask)
            plsc.store_compressed(peid.at[pl.ds(m, 16)], eid16, mask=mask)
            m = m + plsc.all_reduce_population_count(mask)[0]
        plsc.store_compressed(psrc.at[pl.ds(m, 16)], zi, mask=allm)
        plsc.store_compressed(peid.at[pl.ds(m, 16)], zi, mask=allm)

        def batch(b, _):
            pltpu.async_copy(ef_hbm.at[peid.at[pl.ds(b * 16, 16)]], rowb,
                             gsem).wait()
            svec = psrc[pl.ds(b * 16, 16)]
            live = (b * 16 + iota) < m
            svec = jnp.where(live, svec, ROWS)
            for l in range(16):
                sloc = svec[l]
                for v in range(16):
                    plsc.addupdate(acc.at[sloc, pl.ds(v * 16, 16)],
                                   rowb[l, pl.ds(v * 16, 16)])
            return 0

        lax.fori_loop(0, (m + 15) // 16, batch, 0)
        return 0

    lax.fori_loop(0, NCHUNK, chunk, 0)
    pltpu.sync_copy(acc.at[pl.ds(0, ROWS)], out_hbm.at[pl.ds(lo, ROWS)])


def _sc_efagg(src_pad, ef):
    kfn = pl.kernel(
        _efagg_body,
        out_type=jax.ShapeDtypeStruct((N_PAD, D), jnp.float32),
        mesh=plsc.VectorSubcoreMesh(**_SC_MESH),
        compiler_params=_SC_PARAMS,
        scratch_types=[
            pltpu.VMEM((CH,), jnp.int32),
            pltpu.VMEM((CH + 48,), jnp.int32),
            pltpu.VMEM((CH + 48,), jnp.int32),
            pltpu.VMEM((16, D), jnp.float32),
            pltpu.VMEM((ROWS + 1, D), jnp.float32),
            pltpu.SemaphoreType.DMA,
        ],
    )
    return kfn(src_pad, ef)


def _epilogue_body(xc_ref, ea_ref, xlin_ref, dinv_ref, wg1_ref, wg2_ref,
                   bg_ref, gamma_ref, beta_ref, out_ref):
    dinv = dinv_ref[...]
    x_conv = xc_ref[...] + (dinv * dinv) * xlin_ref[...]
    ef_agg = ea_ref[...]
    gate = jax.nn.sigmoid(
        jnp.dot(x_conv, wg1_ref[...], preferred_element_type=jnp.float32)
        + jnp.dot(ef_agg, wg2_ref[...], preferred_element_type=jnp.float32)
        + bg_ref[...]
    )
    out = gate * x_conv + (1.0 - gate) * ef_agg
    mu = jnp.mean(out, axis=-1, keepdims=True)
    var = jnp.mean((out - mu) ** 2, axis=-1, keepdims=True)
    out = (out - mu) * lax.rsqrt(var + 1e-5) * gamma_ref[...] + beta_ref[...]
    out_ref[...] = jnp.maximum(out, 0.0)


def _epilogue(xc, ea, x_lin, dinv, W_gate, b_gate, ln_gamma, ln_beta):
    row_spec = pl.BlockSpec((_BN, D), lambda i: (i, 0))
    return pl.pallas_call(
        _epilogue_body,
        grid=(N // _BN,),
        in_specs=[
            row_spec, row_spec, row_spec,
            pl.BlockSpec((_BN, 1), lambda i: (i, 0)),
            pl.BlockSpec((D, D), lambda i: (0, 0)),
            pl.BlockSpec((D, D), lambda i: (0, 0)),
            pl.BlockSpec((1, D), lambda i: (0, 0)),
            pl.BlockSpec((1, D), lambda i: (0, 0)),
            pl.BlockSpec((1, D), lambda i: (0, 0)),
        ],
        out_specs=pl.BlockSpec((_BN, D), lambda i: (i, 0)),
        out_shape=jax.ShapeDtypeStruct((N, D), jnp.float32),
    )(xc, ea, x_lin, dinv.reshape(N_PAD, 1)[:N], W_gate[:D], W_gate[D:],
      b_gate.reshape(1, D), ln_gamma.reshape(1, D), ln_beta.reshape(1, D))


def kernel(x, edge_index, edge_attr, W_conv, b_conv, W_ep, b_ep, W_gate,
           b_gate, ln_gamma, ln_beta):
    pad = E_PAD - E
    src_pad = jnp.concatenate([edge_index[0], jnp.zeros((pad,), jnp.int32)])
    dst_pad = jnp.concatenate([edge_index[1], jnp.zeros((pad,), jnp.int32)])
    ea_pad = jnp.concatenate(
        [edge_attr, jnp.zeros((pad, D_EDGE), jnp.float32)])

    x_lin = _xlin(x, W_conv, b_conv)
    ef, edge_weight = _ef_and_weight(ea_pad, W_ep, b_ep)
    w_pad = edge_weight.reshape(E_PAD)

    deg16 = _sc_degree(dst_pad, w_pad)
    dinv = _dinv(deg16)

    xc = _sc_xconv(src_pad, dst_pad, w_pad, dinv, x_lin)
    ea = _sc_efagg(src_pad, ef)

    return _epilogue(xc[:N], ea[:N], x_lin, dinv, W_gate, b_gate,
                     ln_gamma, ln_beta)
